# baseline (speedup 1.0000x reference)
import functools

import jax
import jax.numpy as jnp
from jax import lax
from jax.experimental import pallas as pl
from jax.experimental.pallas import tpu as pltpu
from jax.experimental.pallas import tpu_sc as plsc

NC = 2
NS = 16
LANES = 16
CHUNK = 128


def _chunks(total, step):
  out = []
  off = 0
  while off < total:
    out.append(min(step, total - off))
    off += step
  return out


NBUF = 2
IGRP = 8


def _make_agg(e_pad, n_acc, h):
  cpt = e_pad // (NC * NS) // CHUNK
  rpt = n_acc // NS
  assert cpt % IGRP == 0
  mesh = plsc.VectorSubcoreMesh(core_axis_name="c", subcore_axis_name="s")

  @functools.partial(
      pl.kernel,
      out_type=jax.ShapeDtypeStruct((NC, n_acc, h), jnp.float32),
      mesh=mesh,
      scratch_types=[
          pltpu.VMEM((2, IGRP, CHUNK), jnp.int32),
          pltpu.VMEM((2, IGRP, CHUNK), jnp.int32),
          pltpu.VMEM((NBUF, CHUNK, h), jnp.float32),
          pltpu.VMEM_SHARED((n_acc, h), jnp.float32),
          pltpu.SemaphoreType.DMA,
          pltpu.SemaphoreType.DMA,
      ],
  )
  def agg(g_hbm, row_hbm, col_hbm, out_hbm, idxr_v, idxc_v, rows_v, acc_sp,
          gsem, ssem):
    c = lax.axis_index("c")
    s = lax.axis_index("s")

    tile_cbase = (c * NS + s) * cpt

    zv = jnp.zeros((LANES,), jnp.float32)

    def zero_body(i, carry):
      for j in range(h // LANES):
        rows_v[0, i, pl.ds(j * LANES, LANES)] = zv
      return carry

    lax.fori_loop(0, CHUNK, zero_body, 0)

    base = s * rpt
    off = 0
    for sz in _chunks(rpt, CHUNK):
      pltpu.sync_copy(rows_v.at[0, pl.ds(0, sz)],
                      acc_sp.at[pl.ds(base + off, sz)])
      off += sz
    plsc.subcore_barrier()

    def fetch_idx_group(g):
      p = g % 2
      pltpu.sync_copy(row_hbm.at[pl.ds(tile_cbase + g * IGRP, IGRP)],
                      idxr_v.at[p])
      pltpu.sync_copy(col_hbm.at[pl.ds(tile_cbase + g * IGRP, IGRP)],
                      idxc_v.at[p])

    def gather(j):
      b, p, ji = j % NBUF, (j // IGRP) % 2, j % IGRP
      return pltpu.async_copy(g_hbm.at[idxr_v.at[p, ji]], rows_v.at[b], gsem)

    def scatter(j):
      b, p, ji = j % NBUF, (j // IGRP) % 2, j % IGRP
      return pltpu.async_copy(rows_v.at[b], acc_sp.at[idxc_v.at[p, ji]],
                              ssem, add=True)

    gd = [None] * cpt
    sd = [None] * cpt
    fetch_idx_group(0)
    gd[0] = gather(0)
    for j in range(1, cpt + 1):
      if j < cpt:
        if j % IGRP == 0:
          fetch_idx_group(j // IGRP)
        if j >= NBUF:
          sd[j - NBUF].wait()
        gd[j] = gather(j)
      gd[j - 1].wait()
      sd[j - 1] = scatter(j - 1)
    sd[cpt - 2].wait()
    sd[cpt - 1].wait()

    plsc.subcore_barrier()

    off = 0
    for sz in _chunks(rpt, CHUNK):
      pltpu.sync_copy(acc_sp.at[pl.ds(base + off, sz)],
                      out_hbm.at[c, pl.ds(base + off, sz)])
      off += sz

  return agg


def _make_deg(e_pad, n_acc):
  cpt = e_pad // (NC * NS) // CHUNK
  rpt = n_acc // NS
  mesh = plsc.VectorSubcoreMesh(core_axis_name="c", subcore_axis_name="s")

  @functools.partial(
      pl.kernel,
      out_type=jax.ShapeDtypeStruct((NC, n_acc, LANES), jnp.float32),
      mesh=mesh,
      scratch_types=[
          pltpu.VMEM((2, IGRP, CHUNK), jnp.int32),
          pltpu.VMEM((CHUNK, LANES), jnp.float32),
          pltpu.VMEM((CHUNK, LANES), jnp.float32),
          pltpu.VMEM_SHARED((n_acc, LANES), jnp.float32),
          pltpu.SemaphoreType.DMA,
      ],
  )
  def deg(col_hbm, out_hbm, idx_v, ones_v, zbuf_v, acc_sp, sem):
    del sem
    c = lax.axis_index("c")
    s = lax.axis_index("s")

    tile_cbase = (c * NS + s) * cpt

    ov = jnp.ones((LANES,), jnp.float32)
    zv = jnp.zeros((LANES,), jnp.float32)

    def fill_body(i, carry):
      ones_v[i] = ov
      zbuf_v[i] = zv
      return carry

    lax.fori_loop(0, CHUNK, fill_body, 0)

    base = s * rpt
    off = 0
    for sz in _chunks(rpt, CHUNK):
      pltpu.sync_copy(zbuf_v.at[pl.ds(0, sz)], acc_sp.at[pl.ds(base + off, sz)])
      off += sz
    plsc.subcore_barrier()

    for g in range(cpt // IGRP):
      p = g % 2
      pltpu.sync_copy(col_hbm.at[pl.ds(tile_cbase + g * IGRP, IGRP)],
                      idx_v.at[p])
      for ji in range(IGRP):
        pltpu.sync_copy(ones_v, acc_sp.at[idx_v.at[p, ji]], add=True)
    plsc.subcore_barrier()

    off = 0
    for sz in _chunks(rpt, CHUNK):
      pltpu.sync_copy(acc_sp.at[pl.ds(base + off, sz)],
                      out_hbm.at[c, pl.ds(base + off, sz)])
      off += sz

  return deg


def _dinv_from_deg(deg2_ref, n):
  d = deg2_ref[0, 0:n, 0:1] + deg2_ref[1, 0:n, 0:1] + 1.0
  return lax.rsqrt(d)


def kernel(x, edge_index, batch, lin_in_W, lin_in_b, gcn_W, gcn_b,
           bn_gamma, bn_beta, out_W1, out_b1, out_W2, out_b2, out_W3, out_b3):
  n, d_in = x.shape
  h = lin_in_W.shape[1]
  e = edge_index.shape[1]
  num_layers = gcn_W.shape[0]
  nb = 16
  cls = out_W3.shape[1]

  e_pad = -(-e // (NC * NS * CHUNK * 8)) * (NC * NS * CHUNK * 8)
  n_acc = -(-(n + 1) // (NS * 8)) * (NS * 8)

  row = edge_index[0]
  col = edge_index[1]
  pad = e_pad - e
  if pad:
    row = jnp.concatenate([row, jnp.zeros((pad,), jnp.int32)])
    col = jnp.concatenate([col, jnp.full((pad,), n, jnp.int32)])
  row = row.reshape(e_pad // CHUNK, CHUNK)
  col = col.reshape(e_pad // CHUNK, CHUNK)

  deg_fn = _make_deg(e_pad, n_acc)
  agg_fn = _make_agg(e_pad, n_acc, h)

  deg2 = deg_fn(col)

  def tc0(x_ref, w_ref, b_ref, w0_ref, deg_ref, g_ref):
    h0 = jnp.dot(x_ref[...], w_ref[...],
                 preferred_element_type=jnp.float32) + b_ref[...]
    dinv = _dinv_from_deg(deg_ref, n)
    g_ref[...] = dinv * jnp.dot(h0, w0_ref[...],
                                preferred_element_type=jnp.float32)

  g = pl.pallas_call(
      tc0, out_shape=jax.ShapeDtypeStruct((n, h), jnp.float32))(
          x, lin_in_W, lin_in_b.reshape(1, h), gcn_W[0], deg2)

  def tc_mid(acc_ref, g_ref, deg_ref, b_ref, ga_ref, be_ref, wn_ref, o_ref):
    dinv = _dinv_from_deg(deg_ref, n)
    t = dinv * (acc_ref[0, 0:n, :] + acc_ref[1, 0:n, :] + g_ref[...]) + b_ref[...]
    mean = jnp.mean(t, axis=0, keepdims=True)
    ctr = t - mean
    var = jnp.mean(ctr * ctr, axis=0, keepdims=True)
    hh = jnp.maximum(ctr * lax.rsqrt(var + 1e-5) * ga_ref[...] + be_ref[...],
                     0.0)
    o_ref[...] = dinv * jnp.dot(hh, wn_ref[...],
                                preferred_element_type=jnp.float32)

  for i in range(num_layers - 1):
    acc = agg_fn(g, row, col)
    g = pl.pallas_call(
        tc_mid, out_shape=jax.ShapeDtypeStruct((n, h), jnp.float32))(
            acc, g, deg2, gcn_b[i].reshape(1, h), bn_gamma[i].reshape(1, h),
            bn_beta[i].reshape(1, h), gcn_W[i + 1])

  acc = agg_fn(g, row, col)
  li = num_layers - 1

  def tc_fin(acc_ref, g_ref, deg_ref, b_ref, ga_ref, be_ref, bt_ref,
             w1_ref, b1_ref, w2_ref, b2_ref, w3_ref, b3_ref, o_ref):
    dinv = _dinv_from_deg(deg_ref, n)
    t = dinv * (acc_ref[0, 0:n, :] + acc_ref[1, 0:n, :] + g_ref[...]) + b_ref[...]
    mean = jnp.mean(t, axis=0, keepdims=True)
    ctr = t - mean
    var = jnp.mean(ctr * ctr, axis=0, keepdims=True)
    hh = jnp.maximum(ctr * lax.rsqrt(var + 1e-5) * ga_ref[...] + be_ref[...],
                     0.0)
    seg = lax.broadcasted_iota(jnp.int32, (nb, n), 0)
    onehot = (bt_ref[...] == seg).astype(jnp.float32)
    sums = jnp.dot(onehot, hh, preferred_element_type=jnp.float32)
    cnt = jnp.sum(onehot, axis=1, keepdims=True)
    pooled = sums / jnp.maximum(cnt, 1.0)
    o = jnp.maximum(pooled, 0.0)
    o = jnp.maximum(jnp.dot(o, w1_ref[...],
                            preferred_element_type=jnp.float32) + b1_ref[...],
                    0.0)
    o = jnp.maximum(jnp.dot(o, w2_ref[...],
                            preferred_element_type=jnp.float32) + b2_ref[...],
                    0.0)
    o_ref[...] = jnp.dot(o, w3_ref[...],
                         preferred_element_type=jnp.float32) + b3_ref[...]

  out = pl.pallas_call(
      tc_fin, out_shape=jax.ShapeDtypeStruct((nb, cls), jnp.float32))(
          acc, g, deg2, gcn_b[li].reshape(1, h), bn_gamma[li].reshape(1, h),
          bn_beta[li].reshape(1, h), batch.reshape(1, n),
          out_W1, out_b1.reshape(1, -1), out_W2, out_b2.reshape(1, -1),
          out_W3, out_b3.reshape(1, -1))
  return out


# speedup vs baseline: 3.3522x; 3.3522x over previous
import functools

import jax
import jax.numpy as jnp
from jax import lax
from jax.experimental import pallas as pl
from jax.experimental.pallas import tpu as pltpu
from jax.experimental.pallas import tpu_sc as plsc

NC = 2
NS = 16
LANES = 16
CHUNK = 128


def _chunks(total, step):
  out = []
  off = 0
  while off < total:
    out.append(min(step, total - off))
    off += step
  return out


NBUF = 2
IGRP = 8


def _make_agg(e_pad, n_acc, h):
  cpt = e_pad // (NC * NS) // CHUNK
  rpt = n_acc // NS
  assert cpt % IGRP == 0
  mesh = plsc.VectorSubcoreMesh(core_axis_name="c", subcore_axis_name="s")

  @functools.partial(
      pl.kernel,
      out_type=jax.ShapeDtypeStruct((NC, n_acc, h), jnp.float32),
      mesh=mesh,
      scratch_types=[
          pltpu.VMEM((2, IGRP, CHUNK), jnp.int32),
          pltpu.VMEM((2, IGRP, CHUNK), jnp.int32),
          pltpu.VMEM((NBUF, CHUNK, h), jnp.float32),
          pltpu.VMEM_SHARED((n_acc, h), jnp.float32),
          pltpu.SemaphoreType.DMA,
          pltpu.SemaphoreType.DMA,
      ],
  )
  def agg(g_hbm, row_hbm, col_hbm, out_hbm, idxr_v, idxc_v, rows_v, acc_sp,
          gsem, ssem):
    c = lax.axis_index("c")
    s = lax.axis_index("s")

    tile_cbase = (c * NS + s) * cpt

    zv = jnp.zeros((LANES,), jnp.float32)

    def zero_body(i, carry):
      for j in range(h // LANES):
        rows_v[0, i, pl.ds(j * LANES, LANES)] = zv
      return carry

    lax.fori_loop(0, CHUNK, zero_body, 0)

    base = s * rpt
    off = 0
    for sz in _chunks(rpt, CHUNK):
      pltpu.sync_copy(rows_v.at[0, pl.ds(0, sz)],
                      acc_sp.at[pl.ds(base + off, sz)])
      off += sz
    plsc.subcore_barrier()

    def fetch_idx_group(g):
      p = g % 2
      pltpu.sync_copy(row_hbm.at[pl.ds(tile_cbase + g * IGRP, IGRP)],
                      idxr_v.at[p])
      pltpu.sync_copy(col_hbm.at[pl.ds(tile_cbase + g * IGRP, IGRP)],
                      idxc_v.at[p])

    def gather(j):
      b, p, ji = j % NBUF, (j // IGRP) % 2, j % IGRP
      return pltpu.async_copy(g_hbm.at[idxr_v.at[p, ji]], rows_v.at[b], gsem)

    def scatter(j):
      b, p, ji = j % NBUF, (j // IGRP) % 2, j % IGRP
      return pltpu.async_copy(rows_v.at[b], acc_sp.at[idxc_v.at[p, ji]],
                              ssem, add=True)

    gd = [None] * cpt
    sd = [None] * cpt
    fetch_idx_group(0)
    gd[0] = gather(0)
    for j in range(1, cpt + 1):
      if j < cpt:
        if j % IGRP == 0:
          fetch_idx_group(j // IGRP)
        if j >= NBUF:
          sd[j - NBUF].wait()
        gd[j] = gather(j)
      gd[j - 1].wait()
      sd[j - 1] = scatter(j - 1)
    sd[cpt - 2].wait()
    sd[cpt - 1].wait()

    plsc.subcore_barrier()

    off = 0
    for sz in _chunks(rpt, CHUNK):
      pltpu.sync_copy(acc_sp.at[pl.ds(base + off, sz)],
                      out_hbm.at[c, pl.ds(base + off, sz)])
      off += sz

  return agg


def _make_deg(e_pad, n_acc):
  cpt = e_pad // (NC * NS) // CHUNK
  rpt = n_acc // NS
  mesh = plsc.VectorSubcoreMesh(core_axis_name="c", subcore_axis_name="s")

  @functools.partial(
      pl.kernel,
      out_type=jax.ShapeDtypeStruct((NC, n_acc, LANES), jnp.float32),
      mesh=mesh,
      scratch_types=[
          pltpu.VMEM((2, IGRP, CHUNK), jnp.int32),
          pltpu.VMEM((CHUNK, LANES), jnp.float32),
          pltpu.VMEM((CHUNK, LANES), jnp.float32),
          pltpu.VMEM_SHARED((n_acc, LANES), jnp.float32),
          pltpu.SemaphoreType.DMA,
      ],
  )
  def deg(col_hbm, out_hbm, idx_v, ones_v, zbuf_v, acc_sp, sem):
    del sem
    c = lax.axis_index("c")
    s = lax.axis_index("s")

    tile_cbase = (c * NS + s) * cpt

    ov = jnp.ones((LANES,), jnp.float32)
    zv = jnp.zeros((LANES,), jnp.float32)

    def fill_body(i, carry):
      ones_v[i] = ov
      zbuf_v[i] = zv
      return carry

    lax.fori_loop(0, CHUNK, fill_body, 0)

    base = s * rpt
    off = 0
    for sz in _chunks(rpt, CHUNK):
      pltpu.sync_copy(zbuf_v.at[pl.ds(0, sz)], acc_sp.at[pl.ds(base + off, sz)])
      off += sz
    plsc.subcore_barrier()

    for g in range(cpt // IGRP):
      p = g % 2
      pltpu.sync_copy(col_hbm.at[pl.ds(tile_cbase + g * IGRP, IGRP)],
                      idx_v.at[p])
      for ji in range(IGRP):
        pltpu.sync_copy(ones_v, acc_sp.at[idx_v.at[p, ji]], add=True)
    plsc.subcore_barrier()

    off = 0
    for sz in _chunks(rpt, CHUNK):
      pltpu.sync_copy(acc_sp.at[pl.ds(base + off, sz)],
                      out_hbm.at[c, pl.ds(base + off, sz)])
      off += sz

  return deg


def _dinv_from_deg(deg2_ref, n):
  d = deg2_ref[0, 0:n, 0:1] + deg2_ref[1, 0:n, 0:1] + 1.0
  return lax.rsqrt(d)


def kernel(x, edge_index, batch, lin_in_W, lin_in_b, gcn_W, gcn_b,
           bn_gamma, bn_beta, out_W1, out_b1, out_W2, out_b2, out_W3, out_b3):
  n, d_in = x.shape
  h = lin_in_W.shape[1]
  e = edge_index.shape[1]
  num_layers = gcn_W.shape[0]
  nb = 16
  cls = out_W3.shape[1]

  e_pad = -(-e // (NC * NS * CHUNK * 8)) * (NC * NS * CHUNK * 8)
  n_acc = -(-(n + 1) // (NS * 8)) * (NS * 8)

  row = edge_index[0]
  col = edge_index[1]
  pad = e_pad - e
  if pad:
    filler = jnp.arange(pad, dtype=jnp.int32)
    row = jnp.concatenate([row, filler % n])
    col = jnp.concatenate([col, n + filler % (n_acc - n)])
  row = row.reshape(e_pad // CHUNK, CHUNK)
  col = col.reshape(e_pad // CHUNK, CHUNK)

  deg_fn = _make_deg(e_pad, n_acc)
  agg_fn = _make_agg(e_pad, n_acc, h)

  deg2 = deg_fn(col)

  def tc0(x_ref, w_ref, b_ref, w0_ref, deg_ref, g_ref):
    h0 = jnp.dot(x_ref[...], w_ref[...],
                 preferred_element_type=jnp.float32) + b_ref[...]
    dinv = _dinv_from_deg(deg_ref, n)
    g_ref[...] = dinv * jnp.dot(h0, w0_ref[...],
                                preferred_element_type=jnp.float32)

  g = pl.pallas_call(
      tc0, out_shape=jax.ShapeDtypeStruct((n, h), jnp.float32))(
          x, lin_in_W, lin_in_b.reshape(1, h), gcn_W[0], deg2)

  def tc_mid(acc_ref, g_ref, deg_ref, b_ref, ga_ref, be_ref, wn_ref, o_ref):
    dinv = _dinv_from_deg(deg_ref, n)
    t = dinv * (acc_ref[0, 0:n, :] + acc_ref[1, 0:n, :] + g_ref[...]) + b_ref[...]
    mean = jnp.mean(t, axis=0, keepdims=True)
    ctr = t - mean
    var = jnp.mean(ctr * ctr, axis=0, keepdims=True)
    hh = jnp.maximum(ctr * lax.rsqrt(var + 1e-5) * ga_ref[...] + be_ref[...],
                     0.0)
    o_ref[...] = dinv * jnp.dot(hh, wn_ref[...],
                                preferred_element_type=jnp.float32)

  for i in range(num_layers - 1):
    acc = agg_fn(g, row, col)
    g = pl.pallas_call(
        tc_mid, out_shape=jax.ShapeDtypeStruct((n, h), jnp.float32))(
            acc, g, deg2, gcn_b[i].reshape(1, h), bn_gamma[i].reshape(1, h),
            bn_beta[i].reshape(1, h), gcn_W[i + 1])

  acc = agg_fn(g, row, col)
  li = num_layers - 1

  def tc_fin(acc_ref, g_ref, deg_ref, b_ref, ga_ref, be_ref, bt_ref,
             w1_ref, b1_ref, w2_ref, b2_ref, w3_ref, b3_ref, o_ref):
    dinv = _dinv_from_deg(deg_ref, n)
    t = dinv * (acc_ref[0, 0:n, :] + acc_ref[1, 0:n, :] + g_ref[...]) + b_ref[...]
    mean = jnp.mean(t, axis=0, keepdims=True)
    ctr = t - mean
    var = jnp.mean(ctr * ctr, axis=0, keepdims=True)
    hh = jnp.maximum(ctr * lax.rsqrt(var + 1e-5) * ga_ref[...] + be_ref[...],
                     0.0)
    seg = lax.broadcasted_iota(jnp.int32, (nb, n), 0)
    onehot = (bt_ref[...] == seg).astype(jnp.float32)
    sums = jnp.dot(onehot, hh, preferred_element_type=jnp.float32)
    cnt = jnp.sum(onehot, axis=1, keepdims=True)
    pooled = sums / jnp.maximum(cnt, 1.0)
    o = jnp.maximum(pooled, 0.0)
    o = jnp.maximum(jnp.dot(o, w1_ref[...],
                            preferred_element_type=jnp.float32) + b1_ref[...],
                    0.0)
    o = jnp.maximum(jnp.dot(o, w2_ref[...],
                            preferred_element_type=jnp.float32) + b2_ref[...],
                    0.0)
    o_ref[...] = jnp.dot(o, w3_ref[...],
                         preferred_element_type=jnp.float32) + b3_ref[...]

  out = pl.pallas_call(
      tc_fin, out_shape=jax.ShapeDtypeStruct((nb, cls), jnp.float32))(
          acc, g, deg2, gcn_b[li].reshape(1, h), bn_gamma[li].reshape(1, h),
          bn_beta[li].reshape(1, h), batch.reshape(1, n),
          out_W1, out_b1.reshape(1, -1), out_W2, out_b2.reshape(1, -1),
          out_W3, out_b3.reshape(1, -1))
  return out


# speedup vs baseline: 3.4430x; 1.0271x over previous
import functools

import jax
import jax.numpy as jnp
from jax import lax
from jax.experimental import pallas as pl
from jax.experimental.pallas import tpu as pltpu
from jax.experimental.pallas import tpu_sc as plsc

NC = 2
NS = 16
LANES = 16
CHUNK = 128

NBUF = 2
IGRP = 16


def _chunks(total, step):
  out = []
  off = 0
  while off < total:
    out.append(min(step, total - off))
    off += step
  return out


def _make_agg(e_pad, n_acc, h):
  cpt = e_pad // (NC * NS) // CHUNK
  rpt = n_acc // NS
  ngrp = cpt // IGRP
  assert cpt % IGRP == 0 and NBUF <= IGRP
  mesh = plsc.VectorSubcoreMesh(core_axis_name="c", subcore_axis_name="s")

  @functools.partial(
      pl.kernel,
      out_type=jax.ShapeDtypeStruct((NC, n_acc, h), jnp.float32),
      mesh=mesh,
      scratch_types=[
          pltpu.VMEM((2, IGRP, CHUNK), jnp.int32),
          pltpu.VMEM((2, IGRP, CHUNK), jnp.int32),
          pltpu.VMEM((NBUF, CHUNK, h), jnp.float32),
          pltpu.VMEM_SHARED((n_acc, h), jnp.float32),
          pltpu.SemaphoreType.DMA,
          pltpu.SemaphoreType.DMA,
      ],
  )
  def agg(g_hbm, row_hbm, col_hbm, out_hbm, idxr_v, idxc_v, rows_v, acc_sp,
          gsem, ssem):
    c = lax.axis_index("c")
    s = lax.axis_index("s")

    tile_cbase = (c * NS + s) * cpt

    zv = jnp.zeros((LANES,), jnp.float32)

    def zero_body(i, carry):
      for j in range(h // LANES):
        rows_v[0, i, pl.ds(j * LANES, LANES)] = zv
      return carry

    lax.fori_loop(0, CHUNK, zero_body, 0)

    base = s * rpt
    off = 0
    for sz in _chunks(rpt, CHUNK):
      pltpu.sync_copy(rows_v.at[0, pl.ds(0, sz)],
                      acc_sp.at[pl.ds(base + off, sz)])
      off += sz
    plsc.subcore_barrier()

    def fetch_sync(grp):
      p = grp % 2
      pltpu.sync_copy(row_hbm.at[pl.ds(tile_cbase + grp * IGRP, IGRP)],
                      idxr_v.at[p])
      pltpu.sync_copy(col_hbm.at[pl.ds(tile_cbase + grp * IGRP, IGRP)],
                      idxc_v.at[p])

    def gather(j):
      b, p, ji = j % NBUF, (j // IGRP) % 2, j % IGRP
      return pltpu.async_copy(g_hbm.at[idxr_v.at[p, ji]], rows_v.at[b], gsem)

    def scatter(j):
      b, p, ji = j % NBUF, (j // IGRP) % 2, j % IGRP
      return pltpu.async_copy(rows_v.at[b], acc_sp.at[idxc_v.at[p, ji]],
                              ssem, add=True)

    gd = [None] * cpt
    sd = [None] * cpt

    fetch_sync(0)
    gd[0] = gather(0)
    for j in range(1, cpt + 1):
      if j < cpt:
        grp, jr = divmod(j, IGRP)
        if jr == 0:
          fetch_sync(grp)
        if j >= NBUF:
          sd[j - NBUF].wait()
        gd[j] = gather(j)
      gd[j - 1].wait()
      sd[j - 1] = scatter(j - 1)
    for j in range(max(0, cpt - NBUF), cpt):
      sd[j].wait()

    plsc.subcore_barrier()

    off = 0
    for sz in _chunks(rpt, CHUNK):
      pltpu.sync_copy(acc_sp.at[pl.ds(base + off, sz)],
                      out_hbm.at[c, pl.ds(base + off, sz)])
      off += sz

  return agg


def _make_deg(e_pad, n_acc):
  cpt = e_pad // (NC * NS) // CHUNK
  rpt = n_acc // NS
  mesh = plsc.VectorSubcoreMesh(core_axis_name="c", subcore_axis_name="s")

  @functools.partial(
      pl.kernel,
      out_type=jax.ShapeDtypeStruct((NC, n_acc, LANES), jnp.float32),
      mesh=mesh,
      scratch_types=[
          pltpu.VMEM((2, IGRP, CHUNK), jnp.int32),
          pltpu.VMEM((CHUNK, LANES), jnp.float32),
          pltpu.VMEM((CHUNK, LANES), jnp.float32),
          pltpu.VMEM_SHARED((n_acc, LANES), jnp.float32),
          pltpu.SemaphoreType.DMA,
      ],
  )
  def deg(col_hbm, out_hbm, idx_v, ones_v, zbuf_v, acc_sp, sem):
    del sem
    c = lax.axis_index("c")
    s = lax.axis_index("s")

    tile_cbase = (c * NS + s) * cpt

    ov = jnp.ones((LANES,), jnp.float32)
    zv = jnp.zeros((LANES,), jnp.float32)

    def fill_body(i, carry):
      ones_v[i] = ov
      zbuf_v[i] = zv
      return carry

    lax.fori_loop(0, CHUNK, fill_body, 0)

    base = s * rpt
    off = 0
    for sz in _chunks(rpt, CHUNK):
      pltpu.sync_copy(zbuf_v.at[pl.ds(0, sz)], acc_sp.at[pl.ds(base + off, sz)])
      off += sz
    plsc.subcore_barrier()

    for g in range(cpt // IGRP):
      p = g % 2
      pltpu.sync_copy(col_hbm.at[pl.ds(tile_cbase + g * IGRP, IGRP)],
                      idx_v.at[p])
      for ji in range(IGRP):
        pltpu.sync_copy(ones_v, acc_sp.at[idx_v.at[p, ji]], add=True)
    plsc.subcore_barrier()

    off = 0
    for sz in _chunks(rpt, CHUNK):
      pltpu.sync_copy(acc_sp.at[pl.ds(base + off, sz)],
                      out_hbm.at[c, pl.ds(base + off, sz)])
      off += sz

  return deg


def _dinv_from_deg(deg2_ref, n):
  d = deg2_ref[0, 0:n, 0:1] + deg2_ref[1, 0:n, 0:1] + 1.0
  return lax.rsqrt(d)


def kernel(x, edge_index, batch, lin_in_W, lin_in_b, gcn_W, gcn_b,
           bn_gamma, bn_beta, out_W1, out_b1, out_W2, out_b2, out_W3, out_b3):
  n, d_in = x.shape
  h = lin_in_W.shape[1]
  e = edge_index.shape[1]
  num_layers = gcn_W.shape[0]
  nb = 16
  cls = out_W3.shape[1]

  e_pad = -(-e // (NC * NS * CHUNK * IGRP)) * (NC * NS * CHUNK * IGRP)
  n_acc = -(-(n + 1) // (NS * 8)) * (NS * 8)

  row = edge_index[0]
  col = edge_index[1]
  pad = e_pad - e
  if pad:
    filler = jnp.arange(pad, dtype=jnp.int32)
    row = jnp.concatenate([row, filler % n])
    col = jnp.concatenate([col, n + filler % (n_acc - n)])
  row = row.reshape(e_pad // CHUNK, CHUNK)
  col = col.reshape(e_pad // CHUNK, CHUNK)

  deg_fn = _make_deg(e_pad, n_acc)
  agg_fn = _make_agg(e_pad, n_acc, h)

  deg2 = deg_fn(col)

  def tc0(x_ref, w_ref, b_ref, w0_ref, deg_ref, g_ref):
    h0 = jnp.dot(x_ref[...], w_ref[...],
                 preferred_element_type=jnp.float32) + b_ref[...]
    dinv = _dinv_from_deg(deg_ref, n)
    g_ref[...] = dinv * jnp.dot(h0, w0_ref[...],
                                preferred_element_type=jnp.float32)

  g = pl.pallas_call(
      tc0, out_shape=jax.ShapeDtypeStruct((n, h), jnp.float32))(
          x, lin_in_W, lin_in_b.reshape(1, h), gcn_W[0], deg2)

  def tc_mid(acc_ref, g_ref, deg_ref, b_ref, ga_ref, be_ref, wn_ref, o_ref):
    dinv = _dinv_from_deg(deg_ref, n)
    t = dinv * (acc_ref[0, 0:n, :] + acc_ref[1, 0:n, :] + g_ref[...]) + b_ref[...]
    mean = jnp.mean(t, axis=0, keepdims=True)
    ctr = t - mean
    var = jnp.mean(ctr * ctr, axis=0, keepdims=True)
    hv = jnp.maximum(ctr * lax.rsqrt(var + 1e-5) * ga_ref[...] + be_ref[...],
                     0.0)
    o_ref[...] = dinv * jnp.dot(hv, wn_ref[...],
                                preferred_element_type=jnp.float32)

  for i in range(num_layers - 1):
    acc = agg_fn(g, row, col)
    g = pl.pallas_call(
        tc_mid, out_shape=jax.ShapeDtypeStruct((n, h), jnp.float32))(
            acc, g, deg2, gcn_b[i].reshape(1, h), bn_gamma[i].reshape(1, h),
            bn_beta[i].reshape(1, h), gcn_W[i + 1])

  acc = agg_fn(g, row, col)
  li = num_layers - 1

  def tc_fin(acc_ref, g_ref, deg_ref, b_ref, ga_ref, be_ref, bt_ref,
             w1_ref, b1_ref, w2_ref, b2_ref, w3_ref, b3_ref, o_ref):
    dinv = _dinv_from_deg(deg_ref, n)
    t = dinv * (acc_ref[0, 0:n, :] + acc_ref[1, 0:n, :] + g_ref[...]) + b_ref[...]
    mean = jnp.mean(t, axis=0, keepdims=True)
    ctr = t - mean
    var = jnp.mean(ctr * ctr, axis=0, keepdims=True)
    hv = jnp.maximum(ctr * lax.rsqrt(var + 1e-5) * ga_ref[...] + be_ref[...],
                     0.0)
    seg = lax.broadcasted_iota(jnp.int32, (nb, n), 0)
    onehot = (bt_ref[...] == seg).astype(jnp.float32)
    sums = jnp.dot(onehot, hv, preferred_element_type=jnp.float32)
    cnt = jnp.sum(onehot, axis=1, keepdims=True)
    pooled = sums / jnp.maximum(cnt, 1.0)
    o = jnp.maximum(pooled, 0.0)
    o = jnp.maximum(jnp.dot(o, w1_ref[...],
                            preferred_element_type=jnp.float32) + b1_ref[...],
                    0.0)
    o = jnp.maximum(jnp.dot(o, w2_ref[...],
                            preferred_element_type=jnp.float32) + b2_ref[...],
                    0.0)
    o_ref[...] = jnp.dot(o, w3_ref[...],
                         preferred_element_type=jnp.float32) + b3_ref[...]

  out = pl.pallas_call(
      tc_fin, out_shape=jax.ShapeDtypeStruct((nb, cls), jnp.float32))(
          acc, g, deg2, gcn_b[li].reshape(1, h), bn_gamma[li].reshape(1, h),
          bn_beta[li].reshape(1, h), batch.reshape(1, n),
          out_W1, out_b1.reshape(1, -1), out_W2, out_b2.reshape(1, -1),
          out_W3, out_b3.reshape(1, -1))
  return out


# speedup vs baseline: 3.4597x; 1.0048x over previous
import functools

import jax
import jax.numpy as jnp
from jax import lax
from jax.experimental import pallas as pl
from jax.experimental.pallas import tpu as pltpu
from jax.experimental.pallas import tpu_sc as plsc

NC = 2
NS = 16
LANES = 16
CHUNK = 128

NBUF = 2
IGRP = 16


def _chunks(total, step):
  out = []
  off = 0
  while off < total:
    out.append(min(step, total - off))
    off += step
  return out


def _make_agg(e_pad, n_acc, h):
  cpt = e_pad // (NC * NS) // CHUNK
  rpt = n_acc // NS
  ngrp = cpt // IGRP
  assert cpt % IGRP == 0 and NBUF <= IGRP
  mesh = plsc.VectorSubcoreMesh(core_axis_name="c", subcore_axis_name="s")

  @functools.partial(
      pl.kernel,
      out_type=jax.ShapeDtypeStruct((NC, n_acc, h), jnp.float32),
      mesh=mesh,
      scratch_types=[
          pltpu.VMEM((2, IGRP, CHUNK), jnp.int32),
          pltpu.VMEM((2, IGRP, CHUNK), jnp.int32),
          pltpu.VMEM((NBUF, CHUNK, h), jnp.float32),
          pltpu.VMEM_SHARED((n_acc, h), jnp.float32),
          pltpu.SemaphoreType.DMA,
          pltpu.SemaphoreType.DMA,
      ],
  )
  def agg(g_hbm, row_hbm, col_hbm, out_hbm, idxr_v, idxc_v, rows_v, acc_sp,
          gsem, ssem):
    c = lax.axis_index("c")
    s = lax.axis_index("s")

    tile_cbase = (c * NS + s) * cpt

    zv = jnp.zeros((LANES,), jnp.float32)

    def zero_body(i, carry):
      for j in range(h // LANES):
        rows_v[1, i, pl.ds(j * LANES, LANES)] = zv
      return carry

    lax.fori_loop(0, CHUNK, zero_body, 0)

    def fetch_sync(grp):
      p = grp % 2
      pltpu.sync_copy(row_hbm.at[pl.ds(tile_cbase + grp * IGRP, IGRP)],
                      idxr_v.at[p])
      pltpu.sync_copy(col_hbm.at[pl.ds(tile_cbase + grp * IGRP, IGRP)],
                      idxc_v.at[p])

    def gather(j):
      b, p, ji = j % NBUF, (j // IGRP) % 2, j % IGRP
      return pltpu.async_copy(g_hbm.at[idxr_v.at[p, ji]], rows_v.at[b], gsem)

    def scatter(j):
      b, p, ji = j % NBUF, (j // IGRP) % 2, j % IGRP
      return pltpu.async_copy(rows_v.at[b], acc_sp.at[idxc_v.at[p, ji]],
                              ssem, add=True)

    gd = [None] * cpt
    sd = [None] * cpt

    fetch_sync(0)
    gd[0] = gather(0)

    base = s * rpt
    off = 0
    for sz in _chunks(rpt, CHUNK):
      pltpu.sync_copy(rows_v.at[1, pl.ds(0, sz)],
                      acc_sp.at[pl.ds(base + off, sz)])
      off += sz
    plsc.subcore_barrier()

    for j in range(1, cpt + 1):
      if j < cpt:
        grp, jr = divmod(j, IGRP)
        if jr == 0:
          fetch_sync(grp)
        if j >= NBUF:
          sd[j - NBUF].wait()
        gd[j] = gather(j)
      gd[j - 1].wait()
      sd[j - 1] = scatter(j - 1)
    for j in range(max(0, cpt - NBUF), cpt):
      sd[j].wait()

    plsc.subcore_barrier()

    off = 0
    for sz in _chunks(rpt, CHUNK):
      pltpu.sync_copy(acc_sp.at[pl.ds(base + off, sz)],
                      out_hbm.at[c, pl.ds(base + off, sz)])
      off += sz

  return agg


def _make_deg(e_pad, n_acc):
  cpt = e_pad // (NC * NS) // CHUNK
  rpt = n_acc // NS
  mesh = plsc.VectorSubcoreMesh(core_axis_name="c", subcore_axis_name="s")

  @functools.partial(
      pl.kernel,
      out_type=jax.ShapeDtypeStruct((NC, n_acc, LANES), jnp.float32),
      mesh=mesh,
      scratch_types=[
          pltpu.VMEM((2, IGRP, CHUNK), jnp.int32),
          pltpu.VMEM((CHUNK, LANES), jnp.float32),
          pltpu.VMEM((CHUNK, LANES), jnp.float32),
          pltpu.VMEM_SHARED((n_acc, LANES), jnp.float32),
          pltpu.SemaphoreType.DMA,
      ],
  )
  def deg(col_hbm, out_hbm, idx_v, ones_v, zbuf_v, acc_sp, sem):
    del sem
    c = lax.axis_index("c")
    s = lax.axis_index("s")

    tile_cbase = (c * NS + s) * cpt

    ov = jnp.ones((LANES,), jnp.float32)
    zv = jnp.zeros((LANES,), jnp.float32)

    def fill_body(i, carry):
      ones_v[i] = ov
      zbuf_v[i] = zv
      return carry

    lax.fori_loop(0, CHUNK, fill_body, 0)

    base = s * rpt
    off = 0
    for sz in _chunks(rpt, CHUNK):
      pltpu.sync_copy(zbuf_v.at[pl.ds(0, sz)], acc_sp.at[pl.ds(base + off, sz)])
      off += sz
    plsc.subcore_barrier()

    for g in range(cpt // IGRP):
      p = g % 2
      pltpu.sync_copy(col_hbm.at[pl.ds(tile_cbase + g * IGRP, IGRP)],
                      idx_v.at[p])
      for ji in range(IGRP):
        pltpu.sync_copy(ones_v, acc_sp.at[idx_v.at[p, ji]], add=True)
    plsc.subcore_barrier()

    off = 0
    for sz in _chunks(rpt, CHUNK):
      pltpu.sync_copy(acc_sp.at[pl.ds(base + off, sz)],
                      out_hbm.at[c, pl.ds(base + off, sz)])
      off += sz

  return deg


def _dinv_from_deg(deg2_ref, n):
  d = deg2_ref[0, 0:n, 0:1] + deg2_ref[1, 0:n, 0:1] + 1.0
  return lax.rsqrt(d)


def kernel(x, edge_index, batch, lin_in_W, lin_in_b, gcn_W, gcn_b,
           bn_gamma, bn_beta, out_W1, out_b1, out_W2, out_b2, out_W3, out_b3):
  n, d_in = x.shape
  h = lin_in_W.shape[1]
  e = edge_index.shape[1]
  num_layers = gcn_W.shape[0]
  nb = 16
  cls = out_W3.shape[1]

  e_pad = -(-e // (NC * NS * CHUNK * IGRP)) * (NC * NS * CHUNK * IGRP)
  n_acc = -(-(n + 1) // (NS * 8)) * (NS * 8)

  row = edge_index[0]
  col = edge_index[1]
  pad = e_pad - e
  if pad:
    filler = jnp.arange(pad, dtype=jnp.int32)
    row = jnp.concatenate([row, filler % n])
    col = jnp.concatenate([col, n + filler % (n_acc - n)])
  row = row.reshape(e_pad // CHUNK, CHUNK)
  col = col.reshape(e_pad // CHUNK, CHUNK)

  deg_fn = _make_deg(e_pad, n_acc)
  agg_fn = _make_agg(e_pad, n_acc, h)

  deg2 = deg_fn(col)

  def tc0(x_ref, w_ref, b_ref, w0_ref, deg_ref, g_ref):
    h0 = jnp.dot(x_ref[...], w_ref[...],
                 preferred_element_type=jnp.float32) + b_ref[...]
    dinv = _dinv_from_deg(deg_ref, n)
    g_ref[...] = dinv * jnp.dot(h0, w0_ref[...],
                                preferred_element_type=jnp.float32)

  g = pl.pallas_call(
      tc0, out_shape=jax.ShapeDtypeStruct((n, h), jnp.float32))(
          x, lin_in_W, lin_in_b.reshape(1, h), gcn_W[0], deg2)

  def tc_mid(acc_ref, g_ref, deg_ref, b_ref, ga_ref, be_ref, wn_ref, o_ref):
    dinv = _dinv_from_deg(deg_ref, n)
    t = dinv * (acc_ref[0, 0:n, :] + acc_ref[1, 0:n, :] + g_ref[...]) + b_ref[...]
    mean = jnp.mean(t, axis=0, keepdims=True)
    ctr = t - mean
    var = jnp.mean(ctr * ctr, axis=0, keepdims=True)
    hv = jnp.maximum(ctr * lax.rsqrt(var + 1e-5) * ga_ref[...] + be_ref[...],
                     0.0)
    o_ref[...] = dinv * jnp.dot(hv, wn_ref[...],
                                preferred_element_type=jnp.float32)

  for i in range(num_layers - 1):
    acc = agg_fn(g, row, col)
    g = pl.pallas_call(
        tc_mid, out_shape=jax.ShapeDtypeStruct((n, h), jnp.float32))(
            acc, g, deg2, gcn_b[i].reshape(1, h), bn_gamma[i].reshape(1, h),
            bn_beta[i].reshape(1, h), gcn_W[i + 1])

  acc = agg_fn(g, row, col)
  li = num_layers - 1

  def tc_fin(acc_ref, g_ref, deg_ref, b_ref, ga_ref, be_ref, bt_ref,
             w1_ref, b1_ref, w2_ref, b2_ref, w3_ref, b3_ref, o_ref):
    dinv = _dinv_from_deg(deg_ref, n)
    t = dinv * (acc_ref[0, 0:n, :] + acc_ref[1, 0:n, :] + g_ref[...]) + b_ref[...]
    mean = jnp.mean(t, axis=0, keepdims=True)
    ctr = t - mean
    var = jnp.mean(ctr * ctr, axis=0, keepdims=True)
    hv = jnp.maximum(ctr * lax.rsqrt(var + 1e-5) * ga_ref[...] + be_ref[...],
                     0.0)
    seg = lax.broadcasted_iota(jnp.int32, (nb, n), 0)
    onehot = (bt_ref[...] == seg).astype(jnp.float32)
    sums = jnp.dot(onehot, hv, preferred_element_type=jnp.float32)
    cnt = jnp.sum(onehot, axis=1, keepdims=True)
    pooled = sums / jnp.maximum(cnt, 1.0)
    o = jnp.maximum(pooled, 0.0)
    o = jnp.maximum(jnp.dot(o, w1_ref[...],
                            preferred_element_type=jnp.float32) + b1_ref[...],
                    0.0)
    o = jnp.maximum(jnp.dot(o, w2_ref[...],
                            preferred_element_type=jnp.float32) + b2_ref[...],
                    0.0)
    o_ref[...] = jnp.dot(o, w3_ref[...],
                         preferred_element_type=jnp.float32) + b3_ref[...]

  out = pl.pallas_call(
      tc_fin, out_shape=jax.ShapeDtypeStruct((nb, cls), jnp.float32))(
          acc, g, deg2, gcn_b[li].reshape(1, h), bn_gamma[li].reshape(1, h),
          bn_beta[li].reshape(1, h), batch.reshape(1, n),
          out_W1, out_b1.reshape(1, -1), out_W2, out_b2.reshape(1, -1),
          out_W3, out_b3.reshape(1, -1))
  return out


# speedup vs baseline: 3.4640x; 1.0013x over previous
import functools

import jax
import jax.numpy as jnp
from jax import lax
from jax.experimental import pallas as pl
from jax.experimental.pallas import tpu as pltpu
from jax.experimental.pallas import tpu_sc as plsc

NC = 2
NS = 16
LANES = 16
CHUNK = 128

NBUF = 2
IGRP = 16


def _chunks(total, step):
  out = []
  off = 0
  while off < total:
    out.append(min(step, total - off))
    off += step
  return out


def _make_agg(e_pad, n_acc, h):
  cpt = e_pad // (NC * NS) // CHUNK
  rpt = n_acc // NS
  ngrp = cpt // IGRP
  assert cpt % IGRP == 0 and NBUF <= IGRP
  mesh = plsc.VectorSubcoreMesh(core_axis_name="c", subcore_axis_name="s")

  @functools.partial(
      pl.kernel,
      out_type=jax.ShapeDtypeStruct((NC, n_acc, h), jnp.float32),
      mesh=mesh,
      scratch_types=[
          pltpu.VMEM((2, IGRP, CHUNK), jnp.int32),
          pltpu.VMEM((2, IGRP, CHUNK), jnp.int32),
          pltpu.VMEM((NBUF, CHUNK, h), jnp.float32),
          pltpu.VMEM_SHARED((n_acc, h), jnp.float32),
          pltpu.SemaphoreType.DMA,
          pltpu.SemaphoreType.DMA,
      ],
  )
  def agg(g_hbm, row_hbm, col_hbm, out_hbm, idxr_v, idxc_v, rows_v, acc_sp,
          gsem, ssem):
    c = lax.axis_index("c")
    s = lax.axis_index("s")

    tile_cbase = (c * NS + s) * cpt

    zv = jnp.zeros((LANES,), jnp.float32)

    def zero_body(i, carry):
      for j in range(h // LANES):
        rows_v[1, i, pl.ds(j * LANES, LANES)] = zv
      return carry

    lax.fori_loop(0, CHUNK, zero_body, 0)

    def fetch_sync(grp):
      p = grp % 2
      pltpu.sync_copy(row_hbm.at[pl.ds(tile_cbase + grp * IGRP, IGRP)],
                      idxr_v.at[p])
      pltpu.sync_copy(col_hbm.at[pl.ds(tile_cbase + grp * IGRP, IGRP)],
                      idxc_v.at[p])

    def gather(j):
      b, p, ji = j % NBUF, (j // IGRP) % 2, j % IGRP
      return pltpu.async_copy(g_hbm.at[idxr_v.at[p, ji]], rows_v.at[b], gsem)

    def scatter(j):
      b, p, ji = j % NBUF, (j // IGRP) % 2, j % IGRP
      return pltpu.async_copy(rows_v.at[b], acc_sp.at[idxc_v.at[p, ji]],
                              ssem, add=True)

    gd = [None] * cpt
    sd = [None] * cpt

    fetch_sync(0)
    gd[0] = gather(0)

    base = s * rpt
    off = 0
    for sz in _chunks(rpt, CHUNK):
      pltpu.sync_copy(rows_v.at[1, pl.ds(0, sz)],
                      acc_sp.at[pl.ds(base + off, sz)])
      off += sz
    plsc.subcore_barrier()

    for j in range(1, cpt + 1):
      if j < cpt:
        grp, jr = divmod(j, IGRP)
        if jr == 0:
          fetch_sync(grp)
        if j >= NBUF:
          sd[j - NBUF].wait()
        gd[j] = gather(j)
      gd[j - 1].wait()
      sd[j - 1] = scatter(j - 1)
    for j in range(max(0, cpt - NBUF), cpt):
      sd[j].wait()

    plsc.subcore_barrier()

    off = 0
    for sz in _chunks(rpt, CHUNK):
      pltpu.sync_copy(acc_sp.at[pl.ds(base + off, sz)],
                      out_hbm.at[c, pl.ds(base + off, sz)])
      off += sz

  return agg


def _make_deg(e_pad, n_acc):
  cpt = e_pad // (NC * NS) // CHUNK
  rpt = n_acc // NS
  mesh = plsc.VectorSubcoreMesh(core_axis_name="c", subcore_axis_name="s")

  @functools.partial(
      pl.kernel,
      out_type=jax.ShapeDtypeStruct((NC, n_acc, LANES), jnp.float32),
      mesh=mesh,
      scratch_types=[
          pltpu.VMEM((2, IGRP, CHUNK), jnp.int32),
          pltpu.VMEM((CHUNK, LANES), jnp.float32),
          pltpu.VMEM((CHUNK, LANES), jnp.float32),
          pltpu.VMEM_SHARED((n_acc, LANES), jnp.float32),
          pltpu.SemaphoreType.DMA,
      ],
  )
  def deg(col_hbm, out_hbm, idx_v, ones_v, zbuf_v, acc_sp, sem):
    del sem
    c = lax.axis_index("c")
    s = lax.axis_index("s")

    tile_cbase = (c * NS + s) * cpt

    ov = jnp.ones((LANES,), jnp.float32)
    zv = jnp.zeros((LANES,), jnp.float32)

    def fill_body(i, carry):
      ones_v[i] = ov
      zbuf_v[i] = zv
      return carry

    lax.fori_loop(0, CHUNK, fill_body, 0)

    base = s * rpt
    off = 0
    for sz in _chunks(rpt, CHUNK):
      pltpu.sync_copy(zbuf_v.at[pl.ds(0, sz)], acc_sp.at[pl.ds(base + off, sz)])
      off += sz
    plsc.subcore_barrier()

    for g in range(cpt // IGRP):
      p = g % 2
      pltpu.sync_copy(col_hbm.at[pl.ds(tile_cbase + g * IGRP, IGRP)],
                      idx_v.at[p])
      for ji in range(IGRP):
        pltpu.sync_copy(ones_v, acc_sp.at[idx_v.at[p, ji]], add=True)
    plsc.subcore_barrier()

    off = 0
    for sz in _chunks(rpt, CHUNK):
      pltpu.sync_copy(acc_sp.at[pl.ds(base + off, sz)],
                      out_hbm.at[c, pl.ds(base + off, sz)])
      off += sz

  return deg


def _dinv_from_deg(deg2_ref, n):
  d = deg2_ref[0, 0:n, 0:1] + deg2_ref[1, 0:n, 0:1] + 1.0
  return lax.rsqrt(d)


def kernel(x, edge_index, batch, lin_in_W, lin_in_b, gcn_W, gcn_b,
           bn_gamma, bn_beta, out_W1, out_b1, out_W2, out_b2, out_W3, out_b3):
  n, d_in = x.shape
  h = lin_in_W.shape[1]
  e = edge_index.shape[1]
  num_layers = gcn_W.shape[0]
  nb = 16
  cls = out_W3.shape[1]

  e_pad = -(-e // (NC * NS * CHUNK * IGRP)) * (NC * NS * CHUNK * IGRP)
  n_acc = -(-(n + 1) // (NS * 8)) * (NS * 8)

  row = edge_index[0]
  col = edge_index[1]
  pad = e_pad - e
  if pad:
    filler = jnp.arange(pad, dtype=jnp.int32)
    row = jnp.concatenate([row, filler % n])
    col = jnp.concatenate([col, n + filler % (n_acc - n)])
  row = row.reshape(e_pad // CHUNK, CHUNK)
  col = col.reshape(e_pad // CHUNK, CHUNK)

  deg_fn = _make_deg(e_pad, n_acc)
  agg_fn = _make_agg(e_pad, n_acc, h)

  deg2 = deg_fn(col)

  def tc0a(x_ref, w_ref, b_ref, w0_ref, gp_ref):
    h0 = jnp.dot(x_ref[...], w_ref[...],
                 preferred_element_type=jnp.float32) + b_ref[...]
    gp_ref[...] = jnp.dot(h0, w0_ref[...],
                          preferred_element_type=jnp.float32)

  def tc0b(gp_ref, deg_ref, g_ref):
    dinv = _dinv_from_deg(deg_ref, n)
    g_ref[...] = dinv * gp_ref[...]

  gp = pl.pallas_call(
      tc0a, out_shape=jax.ShapeDtypeStruct((n, h), jnp.float32))(
          x, lin_in_W, lin_in_b.reshape(1, h), gcn_W[0])
  g = pl.pallas_call(
      tc0b, out_shape=jax.ShapeDtypeStruct((n, h), jnp.float32))(gp, deg2)

  def tc_mid(acc_ref, g_ref, deg_ref, b_ref, ga_ref, be_ref, wn_ref, o_ref):
    dinv = _dinv_from_deg(deg_ref, n)
    t = dinv * (acc_ref[0, 0:n, :] + acc_ref[1, 0:n, :] + g_ref[...]) + b_ref[...]
    mean = jnp.mean(t, axis=0, keepdims=True)
    ctr = t - mean
    var = jnp.mean(ctr * ctr, axis=0, keepdims=True)
    hv = jnp.maximum(ctr * lax.rsqrt(var + 1e-5) * ga_ref[...] + be_ref[...],
                     0.0)
    o_ref[...] = dinv * jnp.dot(hv, wn_ref[...],
                                preferred_element_type=jnp.float32)

  for i in range(num_layers - 1):
    acc = agg_fn(g, row, col)
    g = pl.pallas_call(
        tc_mid, out_shape=jax.ShapeDtypeStruct((n, h), jnp.float32))(
            acc, g, deg2, gcn_b[i].reshape(1, h), bn_gamma[i].reshape(1, h),
            bn_beta[i].reshape(1, h), gcn_W[i + 1])

  acc = agg_fn(g, row, col)
  li = num_layers - 1

  def tc_fin(acc_ref, g_ref, deg_ref, b_ref, ga_ref, be_ref, bt_ref,
             w1_ref, b1_ref, w2_ref, b2_ref, w3_ref, b3_ref, o_ref):
    dinv = _dinv_from_deg(deg_ref, n)
    t = dinv * (acc_ref[0, 0:n, :] + acc_ref[1, 0:n, :] + g_ref[...]) + b_ref[...]
    mean = jnp.mean(t, axis=0, keepdims=True)
    ctr = t - mean
    var = jnp.mean(ctr * ctr, axis=0, keepdims=True)
    hv = jnp.maximum(ctr * lax.rsqrt(var + 1e-5) * ga_ref[...] + be_ref[...],
                     0.0)
    seg = lax.broadcasted_iota(jnp.int32, (nb, n), 0)
    onehot = (bt_ref[...] == seg).astype(jnp.float32)
    sums = jnp.dot(onehot, hv, preferred_element_type=jnp.float32)
    cnt = jnp.sum(onehot, axis=1, keepdims=True)
    pooled = sums / jnp.maximum(cnt, 1.0)
    o = jnp.maximum(pooled, 0.0)
    o = jnp.maximum(jnp.dot(o, w1_ref[...],
                            preferred_element_type=jnp.float32) + b1_ref[...],
                    0.0)
    o = jnp.maximum(jnp.dot(o, w2_ref[...],
                            preferred_element_type=jnp.float32) + b2_ref[...],
                    0.0)
    o_ref[...] = jnp.dot(o, w3_ref[...],
                         preferred_element_type=jnp.float32) + b3_ref[...]

  out = pl.pallas_call(
      tc_fin, out_shape=jax.ShapeDtypeStruct((nb, cls), jnp.float32))(
          acc, g, deg2, gcn_b[li].reshape(1, h), bn_gamma[li].reshape(1, h),
          bn_beta[li].reshape(1, h), batch.reshape(1, n),
          out_W1, out_b1.reshape(1, -1), out_W2, out_b2.reshape(1, -1),
          out_W3, out_b3.reshape(1, -1))
  return out
